# baseline (speedup 1.0000x reference)
import functools

import jax
import jax.numpy as jnp
from jax import lax
from jax.experimental import pallas as pl
from jax.experimental.pallas import tpu as pltpu
from jax.experimental.pallas import tpu_sc as plsc

NC = 2
NS = 16
NW = NC * NS
L = 16

SPC = 32
CN = 128

_params = pltpu.CompilerParams(use_tc_tiling_on_sc=False,
                               needs_layout_passes=False)


def _sc_nodes(nodes, nids, z_n):
  N, DF = nodes.shape
  G = z_n.shape[0]
  GP = G // NS

  n_nc = N // CN
  n_rem = n_nc % NW
  n_base_cnt = n_nc // NW
  n_tail = N - n_nc * CN
  assert n_tail % 8 == 0

  mesh = plsc.VectorSubcoreMesh(core_axis_name="c", subcore_axis_name="s")

  @functools.partial(
      pl.kernel,
      out_type=jax.ShapeDtypeStruct((NC, G, DF), jnp.float32),
      mesh=mesh,
      compiler_params=_params,
      scratch_types=[
          pltpu.VMEM((CN, DF), jnp.float32),
          pltpu.VMEM((CN,), jnp.int32),
          pltpu.VMEM((max(n_tail, 8),), jnp.int32),
          pltpu.VMEM_SHARED((G, DF), jnp.float32),
          pltpu.SemaphoreType.DMA,
      ],
  )
  def k(nodes_h, nids_h, zn_h, outn_h, nbuf, nidx, ntidx, acc_n, sem):
    c = lax.axis_index("c")
    s = lax.axis_index("s")
    w = c * NS + s

    pltpu.sync_copy(zn_h.at[pl.ds(s * GP, GP)], acc_n.at[pl.ds(s * GP, GP)])
    plsc.subcore_barrier()

    def nbody(i, carry):
      base = (w + i * NW) * CN
      d1 = pltpu.async_copy(nodes_h.at[pl.ds(base, CN)], nbuf, sem)
      d2 = pltpu.async_copy(nids_h.at[pl.ds(base, CN)], nidx, sem)
      d1.wait()
      d2.wait()
      pltpu.sync_copy(nbuf, acc_n.at[nidx], add=True)
      return carry

    cnt_n = n_base_cnt + jnp.where(w < n_rem, 1, 0)
    lax.fori_loop(0, cnt_n, nbody, 0)

    if n_tail:
      @pl.when(w == NW - 1)
      def _():
        base = n_nc * CN
        pltpu.sync_copy(nodes_h.at[pl.ds(base, n_tail)],
                        nbuf.at[pl.ds(0, n_tail)])
        pltpu.sync_copy(nids_h.at[pl.ds(base, n_tail)],
                        ntidx.at[pl.ds(0, n_tail)])
        pltpu.sync_copy(nbuf.at[pl.ds(0, n_tail)],
                        acc_n.at[ntidx.at[pl.ds(0, n_tail)]], add=True)

    plsc.subcore_barrier()
    pltpu.sync_copy(acc_n.at[pl.ds(s * GP, GP)],
                    outn_h.at[c, pl.ds(s * GP, GP)])

  return k(nodes, nids, z_n)


def _sc_edges(e3, eids, span_q, z_p, DE):
  FD, S, SW = e3.shape
  FPS = FD // NC
  G = z_p.shape[0]
  GP = G // NS

  n_ec = S // SPC
  e_rem = n_ec % NS
  e_base_cnt = n_ec // NS
  t_spans = S - n_ec * SPC

  mesh = plsc.VectorSubcoreMesh(core_axis_name="c", subcore_axis_name="s")

  @functools.partial(
      pl.kernel,
      out_type=(
          jax.ShapeDtypeStruct((NC, FPS, G, SW), jnp.float32),
          jax.ShapeDtypeStruct((NC, NS, FPS, G), jnp.float32),
      ),
      mesh=mesh,
      compiler_params=_params,
      scratch_types=[
          pltpu.VMEM((FPS, SPC, SW), jnp.float32),
          pltpu.VMEM((SPC,), jnp.int32),
          pltpu.VMEM((t_spans if t_spans else 8,), jnp.int32),
          pltpu.VMEM((SW,), jnp.int32),
          pltpu.VMEM((FPS, G), jnp.float32),
          pltpu.SemaphoreType.DMA,
      ] + [pltpu.VMEM_SHARED((G + 8, SW), jnp.float32) for _ in range(FPS)],
  )
  def k(e3_h, eids_h, spanq_h, zp_h, outp_h, outc_h,
        ebuf, qbuf, qtail, idbuf, acc_ct, sem, *accs):
    c = lax.axis_index("c")
    s = lax.axis_index("s")
    iota = lax.iota(jnp.int32, L)

    for d in range(FPS):
      pltpu.sync_copy(zp_h.at[pl.ds(s * GP, GP)],
                      accs[d].at[pl.ds(s * GP, GP)])

    def zbody(r, carry):
      for d in range(FPS):
        acc_ct[d, pl.ds(r * L, L)] = jnp.zeros((L,), jnp.float32)
      return carry

    lax.fori_loop(0, G // L, zbody, 0)
    plsc.subcore_barrier()

    def fix_mixed(m, kbase, sb):
      def cond(carry):
        return jnp.any(carry)

      def body(carry):
        m = carry
        l = jnp.max(plsc.all_reduce_ffs(m))
        sl = kbase + l
        pltpu.sync_copy(eids_h.at[pl.ds((sb + sl) * SW, SW)], idbuf)
        for b in range(SW // L):
          idv = plsc.load_gather(idbuf.at[:], [b * L + iota])
          for d in range(FPS):
            vals = ebuf[d, sl, pl.ds(b * L, L)]
            plsc.addupdate_scatter(acc_ct.at[d], [idv], vals)
        return m & (iota != l)

      lax.while_loop(cond, body, m)

    def scan_mixed(qref, n, sb):
      for kk in range(pl.cdiv(n, L)):
        base = min(kk * L, n - L)
        qv = qref[pl.ds(base, L)]
        m = qv == G
        if base != kk * L:
          m = m & (iota >= (kk * L - base))
        @pl.when(jnp.any(m))
        def _():
          fix_mixed(m, base, sb)

    def ebody(i, carry):
      sb = (s + i * NS) * SPC
      descs = [pltpu.async_copy(e3_h.at[c * FPS + d, pl.ds(sb, SPC)],
                                ebuf.at[d], sem) for d in range(FPS)]
      dq = pltpu.async_copy(spanq_h.at[pl.ds(sb, SPC)], qbuf, sem)
      for dd in descs:
        dd.wait()
      dq.wait()
      scan_mixed(qbuf, SPC, sb)
      sdescs = [pltpu.async_copy(ebuf.at[d], accs[d].at[qbuf], sem, add=True)
                for d in range(FPS)]
      for dd in sdescs:
        dd.wait()
      return carry

    cnt_e = e_base_cnt + jnp.where(s < e_rem, 1, 0)
    lax.fori_loop(0, cnt_e, ebody, 0)

    if t_spans:
      @pl.when(s == e_rem)
      def _():
        sb = n_ec * SPC
        for d in range(FPS):
          pltpu.sync_copy(e3_h.at[c * FPS + d, pl.ds(sb, t_spans)],
                          ebuf.at[d, pl.ds(0, t_spans)])
        pltpu.sync_copy(spanq_h.at[pl.ds(sb, t_spans)], qtail)
        scan_mixed(qtail, t_spans, sb)
        for d in range(FPS):
          pltpu.sync_copy(ebuf.at[d, pl.ds(0, t_spans)],
                          accs[d].at[qtail], add=True)

    plsc.subcore_barrier()
    for d in range(FPS):
      pltpu.sync_copy(accs[d].at[pl.ds(s * GP, GP)],
                      outp_h.at[c, d, pl.ds(s * GP, GP)])
    pltpu.sync_copy(acc_ct, outc_h.at[c, s])

  return k(e3, eids, span_q, z_p)


def _tc_body(aggp_ref, aggc_ref, aggn_ref, g_ref, w_ref, b_ref, out_ref):
  nc, fps, G, SW = aggp_ref.shape
  de = nc * fps
  df = aggn_ref.shape[2]
  acc_n = aggn_ref[0] + aggn_ref[1]
  out = jnp.dot(g_ref[...], w_ref[de + df:, :],
                preferred_element_type=jnp.float32)
  out += jnp.dot(acc_n, w_ref[de:de + df, :],
                 preferred_element_type=jnp.float32)
  for c in range(nc):
    corr = jnp.sum(aggc_ref[c], axis=0)
    for d in range(fps):
      f = c * fps + d
      colsum = jnp.sum(aggp_ref[c, d], axis=-1) + corr[d]
      out += colsum[:, None] * w_ref[f, :][None, :]
  out_ref[...] = out + b_ref[...]


def kernel(nodes, edges, globals_, node_graph_ids, edge_graph_ids, W, b):
  G, DG = globals_.shape
  E, DE = edges.shape
  DF = nodes.shape[1]
  SW = 128
  S = E // SW
  nids = node_graph_ids.astype(jnp.int32)
  eids = edge_graph_ids.astype(jnp.int32)
  e3 = jnp.transpose(edges).reshape(DE, S, SW)
  first = eids[::SW]
  last = eids[SW - 1::SW]
  span_q = jnp.where(first == last, first, G).astype(jnp.int32)
  z_n = jnp.zeros((G, DF), jnp.float32)
  z_p = jnp.zeros((G, SW), jnp.float32)

  agg_n = _sc_nodes(nodes, nids, z_n)
  agg_p, agg_c = _sc_edges(e3, eids, span_q, z_p, DE)

  out = pl.pallas_call(
      _tc_body,
      out_shape=jax.ShapeDtypeStruct((G, W.shape[1]), jnp.float32),
  )(agg_p, agg_c, agg_n, globals_, W, b.reshape(1, -1))
  return out


# speedup vs baseline: 4.6689x; 4.6689x over previous
import functools

import jax
import jax.numpy as jnp
from jax import lax
from jax.experimental import pallas as pl
from jax.experimental.pallas import tpu as pltpu
from jax.experimental.pallas import tpu_sc as plsc

NC = 2
NS = 16
NW = NC * NS
L = 16

SPC = 32
CN = 128

_params = pltpu.CompilerParams(use_tc_tiling_on_sc=False,
                               needs_layout_passes=False)


def _sc_nodes(nodes, nids, z_n):
  N, DF = nodes.shape
  G = z_n.shape[0]
  GP = G // NS

  n_nc = N // CN
  n_rem = n_nc % NW
  n_base_cnt = n_nc // NW
  n_tail = N - n_nc * CN
  assert n_tail % 8 == 0

  mesh = plsc.VectorSubcoreMesh(core_axis_name="c", subcore_axis_name="s")

  @functools.partial(
      pl.kernel,
      out_type=jax.ShapeDtypeStruct((NC, G, DF), jnp.float32),
      mesh=mesh,
      compiler_params=_params,
      scratch_types=[
          pltpu.VMEM((CN, DF), jnp.float32),
          pltpu.VMEM((CN,), jnp.int32),
          pltpu.VMEM((max(n_tail, 8),), jnp.int32),
          pltpu.VMEM_SHARED((G, DF), jnp.float32),
          pltpu.SemaphoreType.DMA,
      ],
  )
  def k(nodes_h, nids_h, zn_h, outn_h, nbuf, nidx, ntidx, acc_n, sem):
    c = lax.axis_index("c")
    s = lax.axis_index("s")
    w = c * NS + s

    pltpu.sync_copy(zn_h.at[pl.ds(s * GP, GP)], acc_n.at[pl.ds(s * GP, GP)])
    plsc.subcore_barrier()

    def nbody(i, carry):
      base = (w + i * NW) * CN
      d1 = pltpu.async_copy(nodes_h.at[pl.ds(base, CN)], nbuf, sem)
      d2 = pltpu.async_copy(nids_h.at[pl.ds(base, CN)], nidx, sem)
      d1.wait()
      d2.wait()
      pltpu.sync_copy(nbuf, acc_n.at[nidx], add=True)
      return carry

    cnt_n = n_base_cnt + jnp.where(w < n_rem, 1, 0)
    lax.fori_loop(0, cnt_n, nbody, 0)

    if n_tail:
      @pl.when(w == NW - 1)
      def _():
        base = n_nc * CN
        pltpu.sync_copy(nodes_h.at[pl.ds(base, n_tail)],
                        nbuf.at[pl.ds(0, n_tail)])
        pltpu.sync_copy(nids_h.at[pl.ds(base, n_tail)],
                        ntidx.at[pl.ds(0, n_tail)])
        pltpu.sync_copy(nbuf.at[pl.ds(0, n_tail)],
                        acc_n.at[ntidx.at[pl.ds(0, n_tail)]], add=True)

    plsc.subcore_barrier()
    pltpu.sync_copy(acc_n.at[pl.ds(s * GP, GP)],
                    outn_h.at[c, pl.ds(s * GP, GP)])

  return k(nodes, nids, z_n)


def _sc_edges(e4, eids, span_q, z_p):
  _, S, FPS, SW = e4.shape
  G = z_p.shape[0]
  GP = G // NS

  n_ec = S // SPC
  e_rem = n_ec % NS
  e_base_cnt = n_ec // NS
  t_spans = S - n_ec * SPC

  mesh = plsc.VectorSubcoreMesh(core_axis_name="c", subcore_axis_name="s")

  @functools.partial(
      pl.kernel,
      out_type=(
          jax.ShapeDtypeStruct((NC, G, FPS, SW), jnp.float32),
          jax.ShapeDtypeStruct((NC, NS, FPS, G), jnp.float32),
      ),
      mesh=mesh,
      compiler_params=_params,
      scratch_types=[
          pltpu.VMEM((SPC, FPS, SW), jnp.float32),
          pltpu.VMEM((SPC,), jnp.int32),
          pltpu.VMEM((t_spans if t_spans else 8,), jnp.int32),
          pltpu.VMEM((SW,), jnp.int32),
          pltpu.VMEM((FPS, G), jnp.float32),
          pltpu.SemaphoreType.DMA,
          pltpu.VMEM_SHARED((G + 1, FPS, SW), jnp.float32),
      ],
  )
  def k(e4_h, eids_h, spanq_h, zp_h, outp_h, outc_h,
        ebuf, qbuf, qtail, idbuf, acc_ct, sem, acc):
    c = lax.axis_index("c")
    s = lax.axis_index("s")
    iota = lax.iota(jnp.int32, L)

    pltpu.sync_copy(zp_h.at[pl.ds(s * GP, GP)], acc.at[pl.ds(s * GP, GP)])

    def zbody(r, carry):
      for d in range(FPS):
        acc_ct[d, pl.ds(r * L, L)] = jnp.zeros((L,), jnp.float32)
      return carry

    lax.fori_loop(0, G // L, zbody, 0)
    plsc.subcore_barrier()

    def fix_mixed(m, kbase, sb):
      def cond(carry):
        return jnp.any(carry)

      def body(carry):
        m = carry
        l = jnp.max(plsc.all_reduce_ffs(m))
        sl = kbase + l
        pltpu.sync_copy(eids_h.at[pl.ds((sb + sl) * SW, SW)], idbuf)
        for b in range(SW // L):
          idv = plsc.load_gather(idbuf.at[:], [b * L + iota])
          for d in range(FPS):
            vals = ebuf[sl, d, pl.ds(b * L, L)]
            plsc.addupdate_scatter(acc_ct.at[d], [idv], vals)
        return m & (iota != l)

      lax.while_loop(cond, body, m)

    def scan_mixed(qref, n, sb):
      for kk in range(pl.cdiv(n, L)):
        base = min(kk * L, max(n - L, 0))
        qv = qref[pl.ds(base, L)]
        m = qv == G
        if base != kk * L:
          m = m & (iota >= (kk * L - base))
        if n < L:
          m = m & (iota < n)
        @pl.when(jnp.any(m))
        def _():
          fix_mixed(m, base, sb)

    def ebody(i, carry):
      sb = (s + i * NS) * SPC
      d1 = pltpu.async_copy(e4_h.at[c, pl.ds(sb, SPC)], ebuf, sem)
      d2 = pltpu.async_copy(spanq_h.at[pl.ds(sb, SPC)], qbuf, sem)
      d1.wait()
      d2.wait()
      scan_mixed(qbuf, SPC, sb)
      pltpu.sync_copy(ebuf, acc.at[qbuf], add=True)
      return carry

    cnt_e = e_base_cnt + jnp.where(s < e_rem, 1, 0)
    lax.fori_loop(0, cnt_e, ebody, 0)

    if t_spans:
      @pl.when(s == e_rem)
      def _():
        sb = n_ec * SPC
        pltpu.sync_copy(e4_h.at[c, pl.ds(sb, t_spans)],
                        ebuf.at[pl.ds(0, t_spans)])
        pltpu.sync_copy(spanq_h.at[pl.ds(sb, t_spans)], qtail)
        scan_mixed(qtail, t_spans, sb)
        pltpu.sync_copy(ebuf.at[pl.ds(0, t_spans)], acc.at[qtail], add=True)

    plsc.subcore_barrier()
    pltpu.sync_copy(acc.at[pl.ds(s * GP, GP)],
                    outp_h.at[c, pl.ds(s * GP, GP)])
    pltpu.sync_copy(acc_ct, outc_h.at[c, s])

  return k(e4, eids, span_q, z_p)


def _pack_body(in_ref, out_ref):
  ns = out_ref.shape[1]
  for t in range(ns):
    out_ref[0, t] = in_ref[:, pl.ds(t * 128, 128)]


def _tc_pack(et, S, SW):
  DE = et.shape[0]
  FPS = DE // NC
  BS = 100
  return pl.pallas_call(
      _pack_body,
      grid=(NC, S // BS),
      in_specs=[pl.BlockSpec((FPS, BS * SW), lambda c, i: (c, i))],
      out_specs=pl.BlockSpec((1, BS, FPS, SW), lambda c, i: (c, i, 0, 0)),
      out_shape=jax.ShapeDtypeStruct((NC, S, FPS, SW), jnp.float32),
  )(et)


def _tc_body(aggp_ref, aggc_ref, aggn_ref, g_ref, w_ref, b_ref, out_ref):
  nc, G, fps, SW = aggp_ref.shape
  de = nc * fps
  df = aggn_ref.shape[2]
  acc_n = aggn_ref[0] + aggn_ref[1]
  out = jnp.dot(g_ref[...], w_ref[de + df:, :],
                preferred_element_type=jnp.float32)
  out += jnp.dot(acc_n, w_ref[de:de + df, :],
                 preferred_element_type=jnp.float32)
  folded = jnp.zeros((G, de), jnp.float32)
  eye = jnp.eye(de, dtype=jnp.float32)
  for c in range(nc):
    corr = jnp.sum(aggc_ref[c], axis=0)
    for d in range(fps):
      f = c * fps + d
      colsum = jnp.sum(aggp_ref[c, :, d, :], axis=-1) + corr[d]
      folded += colsum[:, None] * eye[f][None, :]
  out += jnp.dot(folded, w_ref[0:de, :], preferred_element_type=jnp.float32)
  out_ref[...] = out + b_ref[...]


def kernel(nodes, edges, globals_, node_graph_ids, edge_graph_ids, W, b):
  G, DG = globals_.shape
  E, DE = edges.shape
  DF = nodes.shape[1]
  SW = 128
  S = E // SW
  nids = node_graph_ids.astype(jnp.int32)
  eids = edge_graph_ids.astype(jnp.int32)
  e4 = _tc_pack(jnp.transpose(edges), S, SW)
  first = eids[::SW]
  last = eids[SW - 1::SW]
  span_q = jnp.where(first == last, first, G).astype(jnp.int32)
  z_n = jnp.zeros((G, DF), jnp.float32)
  z_p = jnp.zeros((G, DE // NC, SW), jnp.float32)

  agg_n = _sc_nodes(nodes, nids, z_n)
  agg_p, agg_c = _sc_edges(e4, eids, span_q, z_p)

  out = pl.pallas_call(
      _tc_body,
      out_shape=jax.ShapeDtypeStruct((G, W.shape[1]), jnp.float32),
  )(agg_p, agg_c, agg_n, globals_, W, b.reshape(1, -1))
  return out


# speedup vs baseline: 5.1935x; 1.1123x over previous
import functools

import jax
import jax.numpy as jnp
from jax import lax
from jax.experimental import pallas as pl
from jax.experimental.pallas import tpu as pltpu
from jax.experimental.pallas import tpu_sc as plsc

NC = 2
NS = 16
NW = NC * NS
L = 16

SPC = 16
CN = 128

_params = pltpu.CompilerParams(use_tc_tiling_on_sc=False,
                               needs_layout_passes=False)


def _sc_nodes(nodes, nids, z_n):
  N, DF = nodes.shape
  G = z_n.shape[0]
  GP = G // NS

  n_nc = N // CN
  n_rem = n_nc % NW
  n_base_cnt = n_nc // NW
  n_tail = N - n_nc * CN
  assert n_tail % 8 == 0

  mesh = plsc.VectorSubcoreMesh(core_axis_name="c", subcore_axis_name="s")

  @functools.partial(
      pl.kernel,
      out_type=jax.ShapeDtypeStruct((NC, G, DF), jnp.float32),
      mesh=mesh,
      compiler_params=_params,
      scratch_types=[
          pltpu.VMEM((CN, DF), jnp.float32),
          pltpu.VMEM((CN,), jnp.int32),
          pltpu.VMEM((max(n_tail, 8),), jnp.int32),
          pltpu.VMEM_SHARED((G, DF), jnp.float32),
          pltpu.SemaphoreType.DMA,
      ],
  )
  def k(nodes_h, nids_h, zn_h, outn_h, nbuf, nidx, ntidx, acc_n, sem):
    c = lax.axis_index("c")
    s = lax.axis_index("s")
    w = c * NS + s

    pltpu.sync_copy(zn_h.at[pl.ds(s * GP, GP)], acc_n.at[pl.ds(s * GP, GP)])
    plsc.subcore_barrier()

    def nbody(i, carry):
      base = (w + i * NW) * CN
      d1 = pltpu.async_copy(nodes_h.at[pl.ds(base, CN)], nbuf, sem)
      d2 = pltpu.async_copy(nids_h.at[pl.ds(base, CN)], nidx, sem)
      d1.wait()
      d2.wait()
      pltpu.sync_copy(nbuf, acc_n.at[nidx], add=True)
      return carry

    cnt_n = n_base_cnt + jnp.where(w < n_rem, 1, 0)
    lax.fori_loop(0, cnt_n, nbody, 0)

    if n_tail:
      @pl.when(w == NW - 1)
      def _():
        base = n_nc * CN
        pltpu.sync_copy(nodes_h.at[pl.ds(base, n_tail)],
                        nbuf.at[pl.ds(0, n_tail)])
        pltpu.sync_copy(nids_h.at[pl.ds(base, n_tail)],
                        ntidx.at[pl.ds(0, n_tail)])
        pltpu.sync_copy(nbuf.at[pl.ds(0, n_tail)],
                        acc_n.at[ntidx.at[pl.ds(0, n_tail)]], add=True)

    plsc.subcore_barrier()
    pltpu.sync_copy(acc_n.at[pl.ds(s * GP, GP)],
                    outn_h.at[c, pl.ds(s * GP, GP)])

  return k(nodes, nids, z_n)


def _sc_edges(e4, eids, span_q, z_p, agg_n):
  _, S, FPS, SW = e4.shape
  G = z_p.shape[0]
  GP = G // NS

  n_ec = S // SPC
  e_rem = n_ec % NS
  e_base_cnt = n_ec // NS
  t_spans = S - n_ec * SPC

  mesh = plsc.VectorSubcoreMesh(core_axis_name="c", subcore_axis_name="s")

  @functools.partial(
      pl.kernel,
      out_type=(
          jax.ShapeDtypeStruct((NC, G, FPS, SW), jnp.float32),
          jax.ShapeDtypeStruct((NC, NS, FPS, G), jnp.float32),
      ),
      mesh=mesh,
      compiler_params=_params,
      scratch_types=[
          pltpu.VMEM((SPC, FPS, SW), jnp.float32),
          pltpu.VMEM((SPC, FPS, SW), jnp.float32),
          pltpu.VMEM((SPC,), jnp.int32),
          pltpu.VMEM((SPC,), jnp.int32),
          pltpu.VMEM((t_spans if t_spans else 8,), jnp.int32),
          pltpu.VMEM((SW,), jnp.int32),
          pltpu.VMEM((FPS, G), jnp.float32),
          pltpu.SemaphoreType.DMA,
          pltpu.SemaphoreType.DMA,
          pltpu.SemaphoreType.DMA,
          pltpu.VMEM_SHARED((G + 1, FPS, SW), jnp.float32),
      ],
  )
  def k(e4_h, eids_h, spanq_h, zp_h, aggn_h, outp_h, outc_h,
        ebuf, ebuf2, qbuf, qbuf2, qtail, idbuf, acc_ct, sem, sem2, ssem,
        acc):
    del aggn_h
    c = lax.axis_index("c")
    s = lax.axis_index("s")
    iota = lax.iota(jnp.int32, L)

    pltpu.sync_copy(zp_h.at[pl.ds(s * GP, GP)], acc.at[pl.ds(s * GP, GP)])

    def zbody(r, carry):
      for d in range(FPS):
        acc_ct[d, pl.ds(r * L, L)] = jnp.zeros((L,), jnp.float32)
      return carry

    lax.fori_loop(0, G // L, zbody, 0)
    plsc.subcore_barrier()

    def fix_mixed(m, kbase, sb, eb):
      def cond(carry):
        return jnp.any(carry)

      def body(carry):
        m = carry
        l = jnp.max(plsc.all_reduce_ffs(m))
        sl = kbase + l
        pltpu.sync_copy(eids_h.at[pl.ds((sb + sl) * SW, SW)], idbuf)
        for b in range(SW // L):
          idv = plsc.load_gather(idbuf.at[:], [b * L + iota])
          for d in range(FPS):
            vals = eb[sl, d, pl.ds(b * L, L)]
            plsc.addupdate_scatter(acc_ct.at[d], [idv], vals)
        return m & (iota != l)

      lax.while_loop(cond, body, m)

    def scan_mixed(qref, n, sb, eb):
      for kk in range(pl.cdiv(n, L)):
        base = min(kk * L, max(n - L, 0))
        qv = qref[pl.ds(base, L)]
        m = qv == G
        if base != kk * L:
          m = m & (iota >= (kk * L - base))
        if n < L:
          m = m & (iota < n)
        @pl.when(jnp.any(m))
        def _():
          fix_mixed(m, base, sb, eb)

    def chunk(sb, eb, qb):
      scan_mixed(qb, SPC, sb, eb)
      return pltpu.async_copy(eb, acc.at[qb], ssem, add=True)

    def loads(sb, eb, qb, sm):
      d1 = pltpu.async_copy(e4_h.at[c, pl.ds(sb, SPC)], eb, sm)
      d2 = pltpu.async_copy(spanq_h.at[pl.ds(sb, SPC)], qb, sm)
      return d1, d2

    def ebody(i, carry):
      sbA = (s + (2 * i) * NS) * SPC
      sbB = (s + (2 * i + 1) * NS) * SPC
      lA = loads(sbA, ebuf, qbuf, sem)
      lB = loads(sbB, ebuf2, qbuf2, sem2)
      for dd in lA:
        dd.wait()
      scA = chunk(sbA, ebuf, qbuf)
      for dd in lB:
        dd.wait()
      scB = chunk(sbB, ebuf2, qbuf2)
      scA.wait()
      scB.wait()
      return carry

    cnt_e = e_base_cnt + jnp.where(s < e_rem, 1, 0)
    lax.fori_loop(0, cnt_e // 2, ebody, 0)

    @pl.when(cnt_e % 2 == 1)
    def _():
      sb = (s + (cnt_e - 1) * NS) * SPC
      lA = loads(sb, ebuf, qbuf, sem)
      for dd in lA:
        dd.wait()
      chunk(sb, ebuf, qbuf).wait()

    if t_spans:
      @pl.when(s == e_rem)
      def _():
        sb = n_ec * SPC
        pltpu.sync_copy(e4_h.at[c, pl.ds(sb, t_spans)],
                        ebuf.at[pl.ds(0, t_spans)])
        pltpu.sync_copy(spanq_h.at[pl.ds(sb, t_spans)], qtail)
        scan_mixed(qtail, t_spans, sb, ebuf)
        pltpu.sync_copy(ebuf.at[pl.ds(0, t_spans)], acc.at[qtail], add=True)

    plsc.subcore_barrier()
    pltpu.sync_copy(acc.at[pl.ds(s * GP, GP)],
                    outp_h.at[c, pl.ds(s * GP, GP)])
    pltpu.sync_copy(acc_ct, outc_h.at[c, s])

  return k(e4, eids, span_q, z_p, agg_n)


def _pack_body(in_ref, out_ref):
  ns = out_ref.shape[1]
  for t in range(ns):
    out_ref[0, t] = in_ref[:, pl.ds(t * 128, 128)]


def _tc_pack(et, S, SW):
  DE = et.shape[0]
  FPS = DE // NC
  BS = 100
  return pl.pallas_call(
      _pack_body,
      grid=(NC, S // BS),
      in_specs=[pl.BlockSpec((FPS, BS * SW), lambda c, i: (c, i))],
      out_specs=pl.BlockSpec((1, BS, FPS, SW), lambda c, i: (c, i, 0, 0)),
      out_shape=jax.ShapeDtypeStruct((NC, S, FPS, SW), jnp.float32),
  )(et)


def _tc_body(aggp_ref, aggc_ref, aggn_ref, g_ref, w_ref, b_ref, out_ref):
  nc, G, fps, SW = aggp_ref.shape
  de = nc * fps
  df = aggn_ref.shape[2]
  acc_n = aggn_ref[0] + aggn_ref[1]
  out = jnp.dot(g_ref[...], w_ref[de + df:, :],
                preferred_element_type=jnp.float32)
  out += jnp.dot(acc_n, w_ref[de:de + df, :],
                 preferred_element_type=jnp.float32)
  folded = jnp.zeros((G, de), jnp.float32)
  eye = jnp.eye(de, dtype=jnp.float32)
  for c in range(nc):
    corr = jnp.sum(aggc_ref[c], axis=0)
    for d in range(fps):
      f = c * fps + d
      colsum = jnp.sum(aggp_ref[c, :, d, :], axis=-1) + corr[d]
      folded += colsum[:, None] * eye[f][None, :]
  out += jnp.dot(folded, w_ref[0:de, :], preferred_element_type=jnp.float32)
  out_ref[...] = out + b_ref[...]


def kernel(nodes, edges, globals_, node_graph_ids, edge_graph_ids, W, b):
  G, DG = globals_.shape
  E, DE = edges.shape
  DF = nodes.shape[1]
  SW = 128
  S = E // SW
  nids = node_graph_ids.astype(jnp.int32)
  eids = edge_graph_ids.astype(jnp.int32)
  e4 = _tc_pack(jnp.transpose(edges), S, SW)
  first = eids[::SW]
  last = eids[SW - 1::SW]
  span_q = jnp.where(first == last, first, G).astype(jnp.int32)
  z_n = jnp.zeros((G, DF), jnp.float32)
  z_p = jnp.zeros((G, DE // NC, SW), jnp.float32)

  agg_n = _sc_nodes(nodes, nids, z_n)
  agg_p, agg_c = _sc_edges(e4, eids, span_q, z_p, agg_n)

  out = pl.pallas_call(
      _tc_body,
      out_shape=jax.ShapeDtypeStruct((G, W.shape[1]), jnp.float32),
  )(agg_p, agg_c, agg_n, globals_, W, b.reshape(1, -1))
  return out
